# baseline (speedup 1.0000x reference)
import functools

import jax
import jax.numpy as jnp
from jax import lax
from jax.experimental import pallas as pl
from jax.experimental.pallas import tpu as pltpu
from jax.experimental.pallas import tpu_sc as plsc

B, N, D = 64, 576, 768
K = 144
PATCH = 16
BB = 8

NC, NS = 2, 16
NW = NC * NS
ROWS = B * K
RPW = ROWS // NW
CH = 32
NCHUNK = RPW // CH


def _topk_idx_kernel(scores_ref, lidx_ref, gidx_ref):
    blk = pl.program_id(0)
    i_sub = lax.broadcasted_iota(jnp.int32, (N, N), 0)
    j_lane = lax.broadcasted_iota(jnp.int32, (N, N), 1)
    eye = (i_sub == j_lane).astype(jnp.float32)
    r_lane = lax.broadcasted_iota(jnp.int32, (N, K), 1)
    q_sub = lax.broadcasted_iota(jnp.int32, (N, K), 0)

    def body(i, _):
        srow = scores_ref[pl.ds(i, 1), :]
        scol = lax.dot_general(eye, srow, (((1,), (1,)), ((), ())),
                               preferred_element_type=jnp.float32)
        a = jnp.broadcast_to(srow, (N, N))
        bm = jnp.broadcast_to(scol, (N, N))
        before = (a > bm) | ((a == bm) & (j_lane < i_sub))
        ranks = jnp.sum(before.astype(jnp.int32), axis=1, keepdims=True)
        hit = jnp.broadcast_to(ranks, (N, K)) == r_lane
        lrow = jnp.sum(jnp.where(hit, q_sub, 0), axis=0, keepdims=True)
        lidx_ref[pl.ds(i, 1), :] = lrow
        gidx_ref[pl.ds(i, 1), :] = lrow + (blk * BB + i) * N
        return 0

    lax.fori_loop(0, BB, body, 0)


def _topk_indices(scores):
    return pl.pallas_call(
        _topk_idx_kernel,
        grid=(B // BB,),
        in_specs=[pl.BlockSpec((BB, N), lambda i: (i, 0))],
        out_specs=[pl.BlockSpec((BB, K), lambda i: (i, 0)),
                   pl.BlockSpec((BB, K), lambda i: (i, 0))],
        out_shape=[jax.ShapeDtypeStruct((B, K), jnp.int32),
                   jax.ShapeDtypeStruct((B, K), jnp.int32)],
    )(scores)


@functools.cache
def _make_sc_gather():
    @functools.partial(
        pl.kernel,
        out_type=jax.ShapeDtypeStruct((ROWS, D), jnp.float32),
        mesh=plsc.VectorSubcoreMesh(core_axis_name="c", subcore_axis_name="s",
                                    num_cores=NC, num_subcores=NS),
        compiler_params=pltpu.CompilerParams(use_tc_tiling_on_sc=True),
        scratch_types=[
            pltpu.VMEM((NCHUNK, CH), jnp.int32),
            pltpu.VMEM((NCHUNK, CH), jnp.int32),
            pltpu.VMEM((CH, D), jnp.float32),
            pltpu.VMEM((CH, D), jnp.float32),
            pltpu.VMEM((CH, D), jnp.float32),
            pltpu.VMEM((CH, D), jnp.float32),
            pltpu.SemaphoreType.DMA,
            pltpu.SemaphoreType.DMA,
            pltpu.SemaphoreType.DMA,
            pltpu.SemaphoreType.DMA,
            pltpu.SemaphoreType.DMA,
            pltpu.SemaphoreType.DMA,
        ],
    )
    def _sc_gather(magno_hbm, pos_hbm, gidx_hbm, lidx_hbm, out_hbm,
                   gi_all, li_all, m0, m1, p0, p1,
                   sem_p0, sem_p1, sem_m0, sem_m1, sem_o0, sem_o1):
        wid = lax.axis_index("s") * NC + lax.axis_index("c")
        wbase = pl.multiple_of(wid * RPW, 8)
        pltpu.sync_copy(gidx_hbm.at[wid], gi_all)
        pltpu.sync_copy(lidx_hbm.at[wid], li_all)

        mbufs = (m0, m1)
        pbufs = (p0, p1)
        sems_p = (sem_p0, sem_p1)
        sems_m = (sem_m0, sem_m1)
        sems_o = (sem_o0, sem_o1)

        def mk(c):
            s = c % 2
            gi = gi_all.at[c]
            li = li_all.at[c]
            pos_d = pltpu.make_async_copy(pos_hbm.at[li], pbufs[s], sems_p[s])
            mag_d = pltpu.make_async_copy(magno_hbm.at[gi], mbufs[s], sems_m[s])
            out_d = pltpu.make_async_copy(
                mbufs[s], out_hbm.at[pl.ds(wbase + c * CH, CH)], sems_o[s])
            return pos_d, mag_d, out_d

        chunks = [mk(c) for c in range(NCHUNK)]
        chunks[0][0].start()
        chunks[0][1].start()
        for c in range(NCHUNK):
            s = c % 2
            pos_d, mag_d, out_d = chunks[c]
            if c + 1 < NCHUNK:
                if c >= 1:
                    chunks[c - 1][2].wait()
                chunks[c + 1][0].start()
                chunks[c + 1][1].start()
            pos_d.wait()
            mag_d.wait()
            mb, pb = mbufs[s], pbufs[s]

            def row(r, _):
                for q in range(D // 16):
                    sl = pl.ds(q * 16, 16)
                    plsc.addupdate(mb.at[r, sl], pb[r, sl])
                return 0

            lax.fori_loop(0, CH, row, 0)
            out_d.start()
        chunks[NCHUNK - 2][2].wait()
        chunks[NCHUNK - 1][2].wait()

    return _sc_gather


def kernel(magno_patches, vit_positional_embedding, line_drawing):
    b, n, d = magno_patches.shape
    p = PATCH
    x = line_drawing[:, 0, :, :]
    hp = x.shape[1] // p
    wp = x.shape[2] // p
    scores = x.reshape(b, hp, p, wp, p).mean(axis=(2, 4)).reshape(b, hp * wp)

    lidx, gidx = _topk_indices(scores)

    magno_flat = magno_patches.reshape(b * n, d)
    pos = vit_positional_embedding[0, 1:, :]
    out = _make_sc_gather()(pos, pos,
                            lidx.reshape(NW, NCHUNK, CH),
                            lidx.reshape(NW, NCHUNK, CH))
    return out.reshape(b, K, d)


# speedup vs baseline: 1.0073x; 1.0073x over previous
import functools

import jax
import jax.numpy as jnp
from jax import lax
from jax.experimental import pallas as pl
from jax.experimental.pallas import tpu as pltpu
from jax.experimental.pallas import tpu_sc as plsc

B, N, D = 64, 576, 768
K = 144
PATCH = 16
BB = 8

NC, NS = 2, 16
NW = NC * NS
ROWS = B * K
RPW = ROWS // NW
CH = 16
NCHUNK = RPW // CH
SPW = B // NW


def _topk_idx_kernel(scores_ref, lidx_ref, gidx_ref):
    blk = pl.program_id(0)
    i_sub = lax.broadcasted_iota(jnp.int32, (N, N), 0)
    j_lane = lax.broadcasted_iota(jnp.int32, (N, N), 1)
    eye = (i_sub == j_lane).astype(jnp.float32)
    r_lane = lax.broadcasted_iota(jnp.int32, (N, K), 1)
    q_sub = lax.broadcasted_iota(jnp.int32, (N, K), 0)

    def body(i, _):
        srow = scores_ref[pl.ds(i, 1), :]
        scol = lax.dot_general(eye, srow, (((1,), (1,)), ((), ())),
                               preferred_element_type=jnp.float32)
        a = jnp.broadcast_to(srow, (N, N))
        bm = jnp.broadcast_to(scol, (N, N))
        before = (a > bm) | ((a == bm) & (j_lane < i_sub))
        ranks = jnp.sum(before.astype(jnp.int32), axis=1, keepdims=True)
        hit = jnp.broadcast_to(ranks, (N, K)) == r_lane
        lrow = jnp.sum(jnp.where(hit, q_sub, 0), axis=0, keepdims=True)
        lidx_ref[pl.ds(i, 1), :] = lrow
        gidx_ref[pl.ds(i, 1), :] = lrow + (blk * BB + i) * N
        return 0

    lax.fori_loop(0, BB, body, 0)


def _topk_indices(scores):
    return pl.pallas_call(
        _topk_idx_kernel,
        grid=(B // BB,),
        in_specs=[pl.BlockSpec((BB, N), lambda i: (i, 0))],
        out_specs=[pl.BlockSpec((BB, K), lambda i: (i, 0)),
                   pl.BlockSpec((BB, K), lambda i: (i, 0))],
        out_shape=[jax.ShapeDtypeStruct((B, K), jnp.int32),
                   jax.ShapeDtypeStruct((B, K), jnp.int32)],
    )(scores)


@functools.cache
def _make_sc_gather():
    @functools.partial(
        pl.kernel,
        out_type=jax.ShapeDtypeStruct((B, K, D), jnp.float32),
        mesh=plsc.VectorSubcoreMesh(core_axis_name="c", subcore_axis_name="s",
                                    num_cores=NC, num_subcores=NS),
        compiler_params=pltpu.CompilerParams(use_tc_tiling_on_sc=True),
        scratch_types=[
            pltpu.VMEM((NCHUNK, CH), jnp.int32),
            pltpu.VMEM((NCHUNK, CH), jnp.int32),
            pltpu.VMEM((CH, D), jnp.float32),
            pltpu.VMEM((CH, D), jnp.float32),
            pltpu.VMEM((CH, D), jnp.float32),
            pltpu.VMEM((CH, D), jnp.float32),
            pltpu.SemaphoreType.DMA,
            pltpu.SemaphoreType.DMA,
            pltpu.SemaphoreType.DMA,
            pltpu.SemaphoreType.DMA,
            pltpu.SemaphoreType.DMA,
            pltpu.SemaphoreType.DMA,
        ],
    )
    def _sc_gather(magno_hbm, pos_hbm, gidx_hbm, lidx_hbm, out_hbm,
                   gi_all, li_all, m0, m1, p0, p1,
                   sem_p0, sem_p1, sem_m0, sem_m1, sem_o0, sem_o1):
        wid = lax.axis_index("s") * NC + lax.axis_index("c")
        wbase = pl.multiple_of(wid * RPW, 8)
        pltpu.sync_copy(gidx_hbm.at[wid], gi_all)
        pltpu.sync_copy(lidx_hbm.at[wid], li_all)

        mbufs = (m0, m1)
        pbufs = (p0, p1)
        sems_p = (sem_p0, sem_p1)
        sems_m = (sem_m0, sem_m1)
        sems_o = (sem_o0, sem_o1)

        kpc = K // CH

        def mk(c):
            s = c % 2
            gi = gi_all.at[c]
            li = li_all.at[c]
            pos_d = pltpu.make_async_copy(pos_hbm.at[li], pbufs[s], sems_p[s])
            mag_d = pltpu.make_async_copy(magno_hbm.at[gi], mbufs[s], sems_m[s])
            sample = wid * SPW + c // kpc
            roff = (c % kpc) * CH
            out_d = pltpu.make_async_copy(
                mbufs[s], out_hbm.at[sample, pl.ds(roff, CH)], sems_o[s])
            return pos_d, mag_d, out_d

        chunks = [mk(c) for c in range(NCHUNK)]
        chunks[0][0].start()
        chunks[0][1].start()
        for c in range(NCHUNK):
            s = c % 2
            pos_d, mag_d, out_d = chunks[c]
            if c + 1 < NCHUNK:
                if c >= 1:
                    chunks[c - 1][2].wait()
                chunks[c + 1][0].start()
                chunks[c + 1][1].start()
            pos_d.wait()
            mag_d.wait()
            mb, pb = mbufs[s], pbufs[s]

            def row(r, _):
                for q in range(D // 16):
                    sl = pl.ds(q * 16, 16)
                    plsc.addupdate(mb.at[r, sl], pb[r, sl])
                return 0

            lax.fori_loop(0, CH, row, 0)
            out_d.start()
        chunks[NCHUNK - 2][2].wait()
        chunks[NCHUNK - 1][2].wait()

    return _sc_gather


def kernel(magno_patches, vit_positional_embedding, line_drawing):
    b, n, d = magno_patches.shape
    p = PATCH
    x = line_drawing[:, 0, :, :]
    hp = x.shape[1] // p
    wp = x.shape[2] // p
    scores = x.reshape(b, hp, p, wp, p).mean(axis=(2, 4)).reshape(b, hp * wp)

    lidx, gidx = _topk_indices(scores)

    magno_flat = magno_patches.reshape(b * n, d)
    pos = vit_positional_embedding[0, 1:, :]
    out = _make_sc_gather()(magno_flat, pos,
                            gidx.reshape(NW, NCHUNK, CH),
                            lidx.reshape(NW, NCHUNK, CH))
    return out


# speedup vs baseline: 1.0831x; 1.0752x over previous
import functools

import jax
import jax.numpy as jnp
from jax import lax
from jax.experimental import pallas as pl
from jax.experimental.pallas import tpu as pltpu
from jax.experimental.pallas import tpu_sc as plsc

B, N, D = 64, 576, 768
K = 144
PATCH = 16
BB = 8

NC, NS = 2, 16
NW = NC * NS
ROWS = B * K
RPW = ROWS // NW
CH = 16
NCHUNK = RPW // CH
SPW = B // NW


IMG = 384
HP = IMG // PATCH


def _mm(a, b):
    return lax.dot_general(a, b, (((1,), (0,)), ((), ())),
                           precision=lax.Precision.HIGHEST,
                           preferred_element_type=jnp.float32)


def _topk_idx_kernel(x_ref, lidx_ref, gidx_ref):
    blk = pl.program_id(0)
    xb = x_ref[...]
    x4 = xb.reshape(BB, HP, PATCH, IMG)
    acc = x4[:, :, 0, :]
    for r in range(1, PATCH):
        acc = acc + x4[:, :, r, :]
    acc = jnp.roll(acc, 8, axis=-1)
    for r in range(PATCH):
        acc = acc + x4[:, :, r, :]
    t = acc
    for sh in (4, 2, 1):
        t = t + jnp.roll(t, -sh, axis=-1)
    s_m = t * jnp.float32(1.0 / 256.0)

    esel = (lax.broadcasted_iota(jnp.int32, (IMG, HP), 0)
            == 16 * lax.broadcasted_iota(jnp.int32, (IMG, HP), 1) + 8
            ).astype(jnp.float32)
    tmat = (lax.broadcasted_iota(jnp.int32, (HP, N), 1) % HP
            == lax.broadcasted_iota(jnp.int32, (HP, N), 0)
            ).astype(jnp.float32)
    maskz = (lax.broadcasted_iota(jnp.int32, (HP, N), 1) // HP
             == lax.broadcasted_iota(jnp.int32, (HP, N), 0))
    ones24 = jnp.full((1, HP), 1.0, dtype=jnp.float32)

    i_sub = lax.broadcasted_iota(jnp.int32, (N, N), 0)
    j_lane = lax.broadcasted_iota(jnp.int32, (N, N), 1)
    eye = (i_sub == j_lane).astype(jnp.float32)
    r_lane = lax.broadcasted_iota(jnp.int32, (N, K), 1)
    q_sub = lax.broadcasted_iota(jnp.int32, (N, K), 0)

    def body(i):
        sm_i = s_m[i]
        s24 = _mm(sm_i, esel)
        z = jnp.where(maskz, _mm(s24, tmat), 0.0)
        srow = _mm(ones24, z)
        scol = lax.dot_general(eye, srow, (((1,), (1,)), ((), ())),
                               precision=lax.Precision.HIGHEST,
                               preferred_element_type=jnp.float32)
        a = jnp.broadcast_to(srow, (N, N))
        bm = jnp.broadcast_to(scol, (N, N))
        before = (a > bm) | ((a == bm) & (j_lane < i_sub))
        ranks = jnp.sum(before.astype(jnp.int32), axis=1, keepdims=True)
        hit = jnp.broadcast_to(ranks, (N, K)) == r_lane
        lrow = jnp.sum(jnp.where(hit, q_sub, 0), axis=0, keepdims=True)
        lidx_ref[pl.ds(i, 1), :] = lrow
        gidx_ref[pl.ds(i, 1), :] = lrow + (blk * BB + i) * N

    for i in range(BB):
        body(i)


def _topk_indices(x):
    return pl.pallas_call(
        _topk_idx_kernel,
        grid=(B // BB,),
        in_specs=[pl.BlockSpec((BB, IMG, IMG), lambda i: (i, 0, 0))],
        out_specs=[pl.BlockSpec((BB, K), lambda i: (i, 0)),
                   pl.BlockSpec((BB, K), lambda i: (i, 0))],
        out_shape=[jax.ShapeDtypeStruct((B, K), jnp.int32),
                   jax.ShapeDtypeStruct((B, K), jnp.int32)],
    )(x)


@functools.cache
def _make_sc_gather():
    @functools.partial(
        pl.kernel,
        out_type=jax.ShapeDtypeStruct((B, K, D), jnp.float32),
        mesh=plsc.VectorSubcoreMesh(core_axis_name="c", subcore_axis_name="s",
                                    num_cores=NC, num_subcores=NS),
        compiler_params=pltpu.CompilerParams(use_tc_tiling_on_sc=True),
        scratch_types=[
            pltpu.VMEM((NCHUNK, CH), jnp.int32),
            pltpu.VMEM((NCHUNK, CH), jnp.int32),
            pltpu.VMEM((CH, D), jnp.float32),
            pltpu.VMEM((CH, D), jnp.float32),
            pltpu.VMEM((CH, D), jnp.float32),
            pltpu.VMEM((CH, D), jnp.float32),
            pltpu.SemaphoreType.DMA,
            pltpu.SemaphoreType.DMA,
            pltpu.SemaphoreType.DMA,
            pltpu.SemaphoreType.DMA,
            pltpu.SemaphoreType.DMA,
            pltpu.SemaphoreType.DMA,
        ],
    )
    def _sc_gather(magno_hbm, pos_hbm, gidx_hbm, lidx_hbm, out_hbm,
                   gi_all, li_all, m0, m1, p0, p1,
                   sem_p0, sem_p1, sem_m0, sem_m1, sem_o0, sem_o1):
        wid = lax.axis_index("s") * NC + lax.axis_index("c")
        wbase = pl.multiple_of(wid * RPW, 8)
        pltpu.sync_copy(gidx_hbm.at[wid], gi_all)
        pltpu.sync_copy(lidx_hbm.at[wid], li_all)

        mbufs = (m0, m1)
        pbufs = (p0, p1)
        sems_p = (sem_p0, sem_p1)
        sems_m = (sem_m0, sem_m1)
        sems_o = (sem_o0, sem_o1)

        kpc = K // CH

        def mk(c):
            s = c % 2
            gi = gi_all.at[c]
            li = li_all.at[c]
            pos_d = pltpu.make_async_copy(pos_hbm.at[li], pbufs[s], sems_p[s])
            mag_d = pltpu.make_async_copy(magno_hbm.at[gi], mbufs[s], sems_m[s])
            sample = wid * SPW + c // kpc
            roff = (c % kpc) * CH
            out_d = pltpu.make_async_copy(
                mbufs[s], out_hbm.at[sample, pl.ds(roff, CH)], sems_o[s])
            return pos_d, mag_d, out_d

        chunks = [mk(c) for c in range(NCHUNK)]
        chunks[0][0].start()
        chunks[0][1].start()
        for c in range(NCHUNK):
            s = c % 2
            pos_d, mag_d, out_d = chunks[c]
            if c + 1 < NCHUNK:
                if c >= 1:
                    chunks[c - 1][2].wait()
                chunks[c + 1][0].start()
                chunks[c + 1][1].start()
            pos_d.wait()
            mag_d.wait()
            mb, pb = mbufs[s], pbufs[s]

            def row(r, _):
                for q in range(D // 16):
                    sl = pl.ds(q * 16, 16)
                    plsc.addupdate(mb.at[r, sl], pb[r, sl])
                return 0

            lax.fori_loop(0, CH, row, 0)
            out_d.start()
        chunks[NCHUNK - 2][2].wait()
        chunks[NCHUNK - 1][2].wait()

    return _sc_gather


def kernel(magno_patches, vit_positional_embedding, line_drawing):
    b, n, d = magno_patches.shape
    x = line_drawing[:, 0, :, :]
    lidx, gidx = _topk_indices(x)

    magno_flat = magno_patches.reshape(b * n, d)
    pos = vit_positional_embedding[0, 1:, :]
    out = _make_sc_gather()(magno_flat, pos,
                            gidx.reshape(NW, NCHUNK, CH),
                            lidx.reshape(NW, NCHUNK, CH))
    return out
